# Initial kernel scaffold; baseline (speedup 1.0000x reference)
import functools

import jax
import jax.numpy as jnp
from jax import lax
from jax.experimental import pallas as pl
from jax.experimental.pallas import tpu as pltpu
from jax.experimental.pallas import tpu_sc as plsc

N = 10000
E = 320000
D = 128
NC = 2
NS = 16
CHUNK = 128
NCHUNKS = E // CHUNK
ROWS_PER_TILE = N // NS
WBLK = 125
P1_BASE, P1_EXTRA = divmod(NCHUNKS, NS)
P2_BASE, P2_EXTRA = divmod(NCHUNKS, NC * NS)

_mesh = plsc.VectorSubcoreMesh(
    core_axis_name="c", subcore_axis_name="s", num_cores=NC, num_subcores=NS)


def _sc_body(x, battr, srcud, dstud, srcb, dstb,
             out_up, out_down, pb,
             sidx, didx, rows, zbuf, acc, gsem):
    c = lax.axis_index("c")
    s = lax.axis_index("s")
    wid = c * NS + s
    row0 = s * ROWS_PER_TILE

    def zrow(i, carry):
        for k in range(D // 16):
            zbuf[i, pl.ds(k * 16, 16)] = jnp.zeros((16,), jnp.float32)
        return carry
    lax.fori_loop(0, WBLK, zrow, 0)

    def zero_acc():
        for t in range(ROWS_PER_TILE // WBLK):
            pltpu.sync_copy(zbuf, acc.at[pl.ds(row0 + t * WBLK, WBLK)])

    def run_edges(src_ref, dst_ref, table_ref, base_edge, nchunks, stride, first):
        def body(j, carry):
            g = first + j * stride
            off = base_edge + g * CHUNK
            pltpu.sync_copy(src_ref.at[pl.ds(off, CHUNK)], sidx)
            pltpu.sync_copy(dst_ref.at[pl.ds(off, CHUNK)], didx)
            pltpu.async_copy(table_ref.at[sidx], rows, gsem).wait()
            pltpu.sync_copy(rows, acc.at[didx], add=True)
            return carry
        lax.fori_loop(0, nchunks, body, 0)

    def write_rows(dst_hbm, dst_row0):
        for t in range(ROWS_PER_TILE // WBLK):
            pltpu.sync_copy(acc.at[pl.ds(row0 + t * WBLK, WBLK)],
                            dst_hbm.at[pl.ds(dst_row0 + t * WBLK, WBLK)])

    zero_acc()
    plsc.subcore_barrier()
    n1 = P1_BASE + jnp.where(s < P1_EXTRA, 1, 0)
    run_edges(srcud, dstud, x, c * E, n1, NS, s)
    plsc.subcore_barrier()

    @pl.when(c == 0)
    def _():
        write_rows(out_up, row0)

    @pl.when(c == 1)
    def _():
        write_rows(out_down, row0)

    zero_acc()
    plsc.subcore_barrier()

    n2 = P2_BASE + jnp.where(wid < P2_EXTRA, 1, 0)
    run_edges(srcb, dstb, battr, 0, n2, NC * NS, wid)
    plsc.subcore_barrier()
    write_rows(pb, c * N + row0)


_sc_call = pl.kernel(
    _sc_body,
    out_type=[
        jax.ShapeDtypeStruct((N, D), jnp.float32),
        jax.ShapeDtypeStruct((N, D), jnp.float32),
        jax.ShapeDtypeStruct((2 * N, D), jnp.float32),
    ],
    mesh=_mesh,
    scratch_types=[
        pltpu.VMEM((CHUNK,), jnp.int32),
        pltpu.VMEM((CHUNK,), jnp.int32),
        pltpu.VMEM((CHUNK, D), jnp.float32),
        pltpu.VMEM((WBLK, D), jnp.float32),
        pltpu.VMEM_SHARED((N, D), jnp.float32),
        pltpu.SemaphoreType.DMA,
    ],
)

_BLK = 1000


def _add_body(a_ref, b_ref, o_ref):
    o_ref[...] = a_ref[...] + b_ref[...]


_tc_add = pl.pallas_call(
    _add_body,
    grid=(N // _BLK,),
    in_specs=[
        pl.BlockSpec((_BLK, D), lambda g: (g, 0)),
        pl.BlockSpec((_BLK, D), lambda g: (g + N // _BLK, 0)),
    ],
    out_specs=pl.BlockSpec((_BLK, D), lambda g: (g, 0)),
    out_shape=jax.ShapeDtypeStruct((N, D), jnp.float32),
)


@jax.jit
def kernel(x, up_index, down_index, boundary_index, boundary_attr):
    srcud = jnp.concatenate([up_index[0], down_index[0]])
    dstud = jnp.concatenate([up_index[1], down_index[1]])
    out_up, out_down, pb = _sc_call(
        x, boundary_attr, srcud, dstud, boundary_index[0], boundary_index[1])
    out_boundary = _tc_add(pb, pb)
    return (out_up, out_down, out_boundary)


# speedup vs baseline: 5.5009x; 5.5009x over previous
import functools

import jax
import jax.numpy as jnp
from jax import lax
from jax.experimental import pallas as pl
from jax.experimental.pallas import tpu as pltpu
from jax.experimental.pallas import tpu_sc as plsc

N = 10000
E = 320000
D = 128
NC = 2
NS = 16
NP = 10240
CHUNK = 128
NCHUNKS = E // CHUNK
ROWS_PER_TILE = NP // NS
WBLK = 128
NBLK = ROWS_PER_TILE // WBLK
P1_BASE, P1_EXTRA = divmod(NCHUNKS, NS)
P2_BASE, P2_EXTRA = divmod(NCHUNKS, NC * NS)

_mesh = plsc.VectorSubcoreMesh(
    core_axis_name="c", subcore_axis_name="s", num_cores=NC, num_subcores=NS)


def _sc_body(x, battr, srcud, dstud, srcb, dstb,
             out_up, out_down, pb,
             sidx, didx, rows, zbuf, acc, gsem):
    c = lax.axis_index("c")
    s = lax.axis_index("s")
    wid = c * NS + s
    row0 = s * ROWS_PER_TILE

    def zrow(i, carry):
        for k in range(D // 16):
            zbuf[i, pl.ds(k * 16, 16)] = jnp.zeros((16,), jnp.float32)
        return carry
    lax.fori_loop(0, WBLK, zrow, 0)

    def zero_acc():
        for t in range(NBLK):
            pltpu.sync_copy(zbuf, acc.at[pl.ds(row0 + t * WBLK, WBLK)])

    def run_edges(src_ref, dst_ref, table_ref, base_edge, nchunks, stride, first):
        def body(j, carry):
            g = first + j * stride
            off = base_edge + g * CHUNK
            pltpu.sync_copy(src_ref.at[pl.ds(off, CHUNK)], sidx)
            pltpu.sync_copy(dst_ref.at[pl.ds(off, CHUNK)], didx)
            pltpu.async_copy(table_ref.at[sidx], rows, gsem).wait()
            pltpu.sync_copy(rows, acc.at[didx], add=True)
            return carry
        lax.fori_loop(0, nchunks, body, 0)

    def write_rows(dst_hbm, dst_row0):
        for t in range(NBLK):
            pltpu.sync_copy(acc.at[pl.ds(row0 + t * WBLK, WBLK)],
                            dst_hbm.at[pl.ds(dst_row0 + t * WBLK, WBLK)])

    zero_acc()
    plsc.subcore_barrier()
    n1 = P1_BASE + jnp.where(s < P1_EXTRA, 1, 0)
    run_edges(srcud, dstud, x, c * E, n1, NS, s)
    plsc.subcore_barrier()

    @pl.when(c == 0)
    def _():
        write_rows(out_up, row0)

    @pl.when(c == 1)
    def _():
        write_rows(out_down, row0)

    zero_acc()
    plsc.subcore_barrier()

    n2 = P2_BASE + jnp.where(wid < P2_EXTRA, 1, 0)
    run_edges(srcb, dstb, battr, 0, n2, NC * NS, wid)
    plsc.subcore_barrier()
    write_rows(pb, c * NP + row0)


_sc_call = pl.kernel(
    _sc_body,
    out_type=[
        jax.ShapeDtypeStruct((NP, D), jnp.float32),
        jax.ShapeDtypeStruct((NP, D), jnp.float32),
        jax.ShapeDtypeStruct((2 * NP, D), jnp.float32),
    ],
    mesh=_mesh,
    scratch_types=[
        pltpu.VMEM((CHUNK,), jnp.int32),
        pltpu.VMEM((CHUNK,), jnp.int32),
        pltpu.VMEM((CHUNK, D), jnp.float32),
        pltpu.VMEM((WBLK, D), jnp.float32),
        pltpu.VMEM_SHARED((NP, D), jnp.float32),
        pltpu.SemaphoreType.DMA,
    ],
)

_BLK = 1024


def _add_body(a_ref, b_ref, o_ref):
    o_ref[...] = a_ref[...] + b_ref[...]


_tc_add = pl.pallas_call(
    _add_body,
    grid=(NP // _BLK,),
    in_specs=[
        pl.BlockSpec((_BLK, D), lambda g: (g, 0)),
        pl.BlockSpec((_BLK, D), lambda g: (g + NP // _BLK, 0)),
    ],
    out_specs=pl.BlockSpec((_BLK, D), lambda g: (g, 0)),
    out_shape=jax.ShapeDtypeStruct((N, D), jnp.float32),
)


@jax.jit
def kernel(x, up_index, down_index, boundary_index, boundary_attr):
    srcud = jnp.concatenate([up_index[0], down_index[0]])
    dstud = jnp.concatenate([up_index[1], down_index[1]])
    out_up, out_down, pb = _sc_call(
        x, boundary_attr, srcud, dstud, boundary_index[0], boundary_index[1])
    out_boundary = _tc_add(pb, pb)
    return (out_up[:N], out_down[:N], out_boundary)


# speedup vs baseline: 7.4678x; 1.3576x over previous
import functools

import jax
import jax.numpy as jnp
from jax import lax
from jax.experimental import pallas as pl
from jax.experimental.pallas import tpu as pltpu
from jax.experimental.pallas import tpu_sc as plsc

N = 10000
E = 320000
D = 128
NC = 2
NS = 16
NP = 10112
CHUNK = 128
ROWS_PER_TILE = NP // NS
WBLKS = (128, 128, 128, 128, 120)

P1_CHUNKS = 2528
E1 = P1_CHUNKS * CHUNK
P1_PAIRS = P1_CHUNKS // NS // 2
P2_CHUNKS = 2560
E2 = P2_CHUNKS * CHUNK
P2_PAIRS = P2_CHUNKS // (NC * NS) // 2

_mesh = plsc.VectorSubcoreMesh(
    core_axis_name="c", subcore_axis_name="s", num_cores=NC, num_subcores=NS)


def _sc_body(x, battr, srcud, dstud, srcb, dstb,
             out_up, out_down, pb,
             sidx0, sidx1, didx0, didx1, rows0, rows1, zbuf, acc,
             isem0, isem1, gsem, ssem):
    c = lax.axis_index("c")
    s = lax.axis_index("s")
    wid = c * NS + s
    row0 = s * ROWS_PER_TILE
    sidxs = (sidx0, sidx1)
    didxs = (didx0, didx1)
    rowss = (rows0, rows1)
    isems = (isem0, isem1)

    def zrow(i, carry):
        for k in range(D // 16):
            zbuf[i, pl.ds(k * 16, 16)] = jnp.zeros((16,), jnp.float32)
        return carry
    lax.fori_loop(0, 128, zrow, 0)

    def zero_acc():
        o = 0
        for w in WBLKS:
            pltpu.sync_copy(zbuf.at[pl.ds(0, w)], acc.at[pl.ds(row0 + o, w)])
            o += w

    def run_edges(src_ref, dst_ref, table_ref, base_edge, npairs, stride, first):
        def off(j):
            return base_edge + (first + j * stride) * CHUNK

        def fire_idx(j, b):
            o = off(j)
            pltpu.async_copy(src_ref.at[pl.ds(o, CHUNK)], sidxs[b], isems[b])
            pltpu.async_copy(dst_ref.at[pl.ds(o, CHUNK)], didxs[b], isems[b])

        def wait_idx(j, b):
            o = off(j)
            pltpu.make_async_copy(src_ref.at[pl.ds(o, CHUNK)], sidxs[b], isems[b]).wait()
            pltpu.make_async_copy(dst_ref.at[pl.ds(o, CHUNK)], didxs[b], isems[b]).wait()

        def wait_scatter(b):
            pltpu.make_async_copy(rowss[b], acc.at[didxs[b]], ssem).wait()

        fire_idx(0, 0)

        def pair(t, carry):
            for b in (0, 1):
                j = 2 * t + b
                nb = 1 - b
                if b == 0:
                    @pl.when(t > 0)
                    def _():
                        wait_scatter(nb)
                else:
                    wait_scatter(nb)
                if b == 0:
                    fire_idx(j + 1, nb)
                else:
                    @pl.when(t < npairs - 1)
                    def _():
                        fire_idx(j + 1, nb)
                wait_idx(j, b)
                pltpu.async_copy(table_ref.at[sidxs[b]], rowss[b], gsem).wait()
                pltpu.async_copy(rowss[b], acc.at[didxs[b]], ssem, add=True)
            return carry

        lax.fori_loop(0, npairs, pair, 0)
        wait_scatter(1)

    def write_rows(dst_hbm, dst_row0):
        o = 0
        for w in WBLKS:
            pltpu.sync_copy(acc.at[pl.ds(row0 + o, w)],
                            dst_hbm.at[pl.ds(dst_row0 + o, w)])
            o += w

    zero_acc()
    plsc.subcore_barrier()
    run_edges(srcud, dstud, x, c * E1, P1_PAIRS, NS, s)
    plsc.subcore_barrier()

    @pl.when(c == 0)
    def _():
        write_rows(out_up, row0)

    @pl.when(c == 1)
    def _():
        write_rows(out_down, row0)

    zero_acc()
    plsc.subcore_barrier()

    run_edges(srcb, dstb, battr, 0, P2_PAIRS, NC * NS, wid)
    plsc.subcore_barrier()
    write_rows(pb, c * NP + row0)


_sc_call = pl.kernel(
    _sc_body,
    out_type=[
        jax.ShapeDtypeStruct((NP, D), jnp.float32),
        jax.ShapeDtypeStruct((NP, D), jnp.float32),
        jax.ShapeDtypeStruct((2 * NP, D), jnp.float32),
    ],
    mesh=_mesh,
    scratch_types=[
        pltpu.VMEM((CHUNK,), jnp.int32),
        pltpu.VMEM((CHUNK,), jnp.int32),
        pltpu.VMEM((CHUNK,), jnp.int32),
        pltpu.VMEM((CHUNK,), jnp.int32),
        pltpu.VMEM((CHUNK, D), jnp.float32),
        pltpu.VMEM((CHUNK, D), jnp.float32),
        pltpu.VMEM((128, D), jnp.float32),
        pltpu.VMEM_SHARED((NP, D), jnp.float32),
        pltpu.SemaphoreType.DMA,
        pltpu.SemaphoreType.DMA,
        pltpu.SemaphoreType.DMA,
        pltpu.SemaphoreType.DMA,
    ],
)

_BLK = NP // 8


def _add_body(a_ref, b_ref, o_ref):
    o_ref[...] = a_ref[...] + b_ref[...]


_tc_add = pl.pallas_call(
    _add_body,
    grid=(NP // _BLK,),
    in_specs=[
        pl.BlockSpec((_BLK, D), lambda g: (g, 0)),
        pl.BlockSpec((_BLK, D), lambda g: (g + NP // _BLK, 0)),
    ],
    out_specs=pl.BlockSpec((_BLK, D), lambda g: (g, 0)),
    out_shape=jax.ShapeDtypeStruct((N, D), jnp.float32),
)


@jax.jit
def kernel(x, up_index, down_index, boundary_index, boundary_attr):
    pad1 = E1 - E
    pad2 = E2 - E
    pmax = max(pad1, pad2)
    pad_src = (jnp.arange(pmax, dtype=jnp.int32) * 37) % N
    pad_dst = N + (jnp.arange(pmax, dtype=jnp.int32) % (NP - N))
    srcud = jnp.concatenate(
        [up_index[0], pad_src[:pad1], down_index[0], pad_src[:pad1]])
    dstud = jnp.concatenate(
        [up_index[1], pad_dst[:pad1], down_index[1], pad_dst[:pad1]])
    srcb = jnp.concatenate([boundary_index[0], pad_src[:pad2]])
    dstb = jnp.concatenate([boundary_index[1], pad_dst[:pad2]])
    out_up, out_down, pb = _sc_call(
        x, boundary_attr, srcud, dstud, srcb, dstb)
    out_boundary = _tc_add(pb, pb)
    return (out_up[:N], out_down[:N], out_boundary)


# speedup vs baseline: 11.9285x; 1.5973x over previous
import functools

import jax
import jax.numpy as jnp
from jax import lax
from jax.experimental import pallas as pl
from jax.experimental.pallas import tpu as pltpu
from jax.experimental.pallas import tpu_sc as plsc

N = 10000
E = 320000
D = 128
NC = 2
NS = 16
NP = 10112
CHUNK = 128
ROWS_PER_TILE = NP // NS
WBLKS = (128, 128, 128, 128, 120)

P1_CHUNKS = 2544
E1 = P1_CHUNKS * CHUNK
P1_OUTER = P1_CHUNKS // NS // 3
P2_CHUNKS = 2592
E2 = P2_CHUNKS * CHUNK
P2_OUTER = P2_CHUNKS // (NC * NS) // 3

_mesh = plsc.VectorSubcoreMesh(
    core_axis_name="c", subcore_axis_name="s", num_cores=NC, num_subcores=NS)


def _sc_body(x, battr, srcud, dstud, srcb, dstb,
             out_up, out_down, pb,
             sidx0, sidx1, sidx2, didx0, didx1, didx2,
             rows0, rows1, rows2, acc,
             isem0, isem1, isem2, gsem0, gsem1, gsem2, ssem0, ssem1, ssem2):
    c = lax.axis_index("c")
    s = lax.axis_index("s")
    wid = c * NS + s
    row0 = s * ROWS_PER_TILE
    sidxs = (sidx0, sidx1, sidx2)
    didxs = (didx0, didx1, didx2)
    rowss = (rows0, rows1, rows2)
    isems = (isem0, isem1, isem2)
    gsems = (gsem0, gsem1, gsem2)
    ssems = (ssem0, ssem1, ssem2)

    def zero_acc():
        def zrow(i, carry):
            for k in range(D // 16):
                rows0[i, pl.ds(k * 16, 16)] = jnp.zeros((16,), jnp.float32)
            return carry
        lax.fori_loop(0, CHUNK, zrow, 0)
        o = 0
        for w in WBLKS:
            pltpu.sync_copy(rows0.at[pl.ds(0, w)], acc.at[pl.ds(row0 + o, w)])
            o += w

    def run_edges(src_ref, dst_ref, table_ref, base_edge, nouter, stride, first):
        def off(j):
            return base_edge + (first + j * stride) * CHUNK

        def fire_idx(j, b):
            o = off(j)
            pltpu.async_copy(src_ref.at[pl.ds(o, CHUNK)], sidxs[b], isems[b])
            pltpu.async_copy(dst_ref.at[pl.ds(o, CHUNK)], didxs[b], isems[b])

        def wait_idx(j, b):
            o = off(j)
            pltpu.make_async_copy(src_ref.at[pl.ds(o, CHUNK)], sidxs[b], isems[b]).wait()
            pltpu.make_async_copy(dst_ref.at[pl.ds(o, CHUNK)], didxs[b], isems[b]).wait()

        def fire_gather(b):
            pltpu.async_copy(table_ref.at[sidxs[b]], rowss[b], gsems[b])

        def wait_gather(b):
            pltpu.make_async_copy(table_ref.at[sidxs[b]], rowss[b], gsems[b]).wait()

        def fire_scatter(b):
            pltpu.async_copy(rowss[b], acc.at[didxs[b]], ssems[b], add=True)

        def wait_scatter(b):
            pltpu.make_async_copy(rowss[b], acc.at[didxs[b]], ssems[b]).wait()

        fire_idx(0, 0)
        wait_idx(0, 0)
        fire_gather(0)
        fire_idx(1, 1)

        def outer(t, carry):
            for b in range(3):
                j = 3 * t + b
                nb = (b + 1) % 3
                pb_ = (b + 2) % 3
                if b < 2:
                    wait_idx(j + 1, nb)
                    fire_gather(nb)
                else:
                    @pl.when(t < nouter - 1)
                    def _():
                        wait_idx(j + 1, nb)
                        fire_gather(nb)
                if b == 0:
                    @pl.when(t > 0)
                    def _():
                        wait_scatter(pb_)
                else:
                    wait_scatter(pb_)
                if b == 0:
                    fire_idx(j + 2, pb_)
                else:
                    @pl.when(t < nouter - 1)
                    def _():
                        fire_idx(j + 2, pb_)
                wait_gather(b)
                fire_scatter(b)
            return carry

        lax.fori_loop(0, nouter, outer, 0)
        wait_scatter(2)

    def write_rows(dst_hbm, dst_row0):
        o = 0
        for w in WBLKS:
            pltpu.sync_copy(acc.at[pl.ds(row0 + o, w)],
                            dst_hbm.at[pl.ds(dst_row0 + o, w)])
            o += w

    zero_acc()
    plsc.subcore_barrier()
    run_edges(srcud, dstud, x, c * E1, P1_OUTER, NS, s)
    plsc.subcore_barrier()

    @pl.when(c == 0)
    def _():
        write_rows(out_up, row0)

    @pl.when(c == 1)
    def _():
        write_rows(out_down, row0)

    zero_acc()
    plsc.subcore_barrier()

    run_edges(srcb, dstb, battr, 0, P2_OUTER, NC * NS, wid)
    plsc.subcore_barrier()
    write_rows(pb, c * NP + row0)


_sc_call = pl.kernel(
    _sc_body,
    out_type=[
        jax.ShapeDtypeStruct((NP, D), jnp.float32),
        jax.ShapeDtypeStruct((NP, D), jnp.float32),
        jax.ShapeDtypeStruct((2 * NP, D), jnp.float32),
    ],
    mesh=_mesh,
    scratch_types=(
        [pltpu.VMEM((CHUNK,), jnp.int32)] * 6
        + [pltpu.VMEM((CHUNK, D), jnp.float32)] * 3
        + [pltpu.VMEM_SHARED((NP, D), jnp.float32)]
        + [pltpu.SemaphoreType.DMA] * 9
    ),
)

_BLK = NP // 8


def _add_body(a_ref, b_ref, o_ref):
    o_ref[...] = a_ref[...] + b_ref[...]


_tc_add = pl.pallas_call(
    _add_body,
    grid=(NP // _BLK,),
    in_specs=[
        pl.BlockSpec((_BLK, D), lambda g: (g, 0)),
        pl.BlockSpec((_BLK, D), lambda g: (g + NP // _BLK, 0)),
    ],
    out_specs=pl.BlockSpec((_BLK, D), lambda g: (g, 0)),
    out_shape=jax.ShapeDtypeStruct((N, D), jnp.float32),
)


@jax.jit
def kernel(x, up_index, down_index, boundary_index, boundary_attr):
    pad1 = E1 - E
    pad2 = E2 - E
    pmax = max(pad1, pad2)
    pad_src = (jnp.arange(pmax, dtype=jnp.int32) * 37) % N
    pad_dst = N + (jnp.arange(pmax, dtype=jnp.int32) % (NP - N))
    srcud = jnp.concatenate(
        [up_index[0], pad_src[:pad1], down_index[0], pad_src[:pad1]])
    dstud = jnp.concatenate(
        [up_index[1], pad_dst[:pad1], down_index[1], pad_dst[:pad1]])
    srcb = jnp.concatenate([boundary_index[0], pad_src[:pad2]])
    dstb = jnp.concatenate([boundary_index[1], pad_dst[:pad2]])
    out_up, out_down, pbp = _sc_call(
        x, boundary_attr, srcud, dstud, srcb, dstb)
    out_boundary = _tc_add(pbp, pbp)
    return (out_up[:N], out_down[:N], out_boundary)


# speedup vs baseline: 13.7366x; 1.1516x over previous
import functools

import jax
import jax.numpy as jnp
from jax import lax
from jax.experimental import pallas as pl
from jax.experimental.pallas import tpu as pltpu
from jax.experimental.pallas import tpu_sc as plsc

N = 10000
E = 320000
D = 128
NC = 2
NS = 16
NP = 10112
CHUNK = 128
NCHUNKS = E // CHUNK
ROWS_PER_TILE = NP // NS
WBLKS = (128, 128, 128, 128, 120)
OUT_BLKS = (128, 128, 128, 128, 8)

P1_OUTER = NCHUNKS // NS // 3
P1_XTRA = NCHUNKS - P1_OUTER * 3 * NS
P2_OUTER = NCHUNKS // (NC * NS) // 3
P2_XTRA = NCHUNKS - P2_OUTER * 3 * NC * NS

_mesh = plsc.VectorSubcoreMesh(
    core_axis_name="c", subcore_axis_name="s", num_cores=NC, num_subcores=NS)


def _sc_body(x, battr, up, down, bnd,
             out_up, out_down, pb,
             sd0, sd1, sd2, rows0, rows1, rows2, acc,
             isem0, isem1, isem2, gsem0, gsem1, gsem2, ssem0, ssem1, ssem2):
    c = lax.axis_index("c")
    s = lax.axis_index("s")
    wid = c * NS + s
    row0 = s * ROWS_PER_TILE
    sds = (sd0, sd1, sd2)
    rowss = (rows0, rows1, rows2)
    isems = (isem0, isem1, isem2)
    gsems = (gsem0, gsem1, gsem2)
    ssems = (ssem0, ssem1, ssem2)

    def zero_acc():
        def zrow(i, carry):
            for k in range(D // 16):
                rows0[i, pl.ds(k * 16, 16)] = jnp.zeros((16,), jnp.float32)
            return carry
        lax.fori_loop(0, CHUNK, zrow, 0)
        o = 0
        for w in WBLKS:
            pltpu.sync_copy(rows0.at[pl.ds(0, w)], acc.at[pl.ds(row0 + o, w)])
            o += w

    def run_edges(idx_ref, table_ref, first, stride, nouter, nxtra):
        def off(j):
            return (first + j * stride) * CHUNK

        def fire_idx(j, b):
            pltpu.async_copy(idx_ref.at[:, pl.ds(off(j), CHUNK)], sds[b], isems[b])

        def wait_idx(j, b):
            pltpu.make_async_copy(
                idx_ref.at[:, pl.ds(off(j), CHUNK)], sds[b], isems[b]).wait()

        def fire_gather(b):
            pltpu.async_copy(table_ref.at[sds[b].at[0]], rowss[b], gsems[b])

        def wait_gather(b):
            pltpu.make_async_copy(table_ref.at[sds[b].at[0]], rowss[b], gsems[b]).wait()

        def fire_scatter(b):
            pltpu.async_copy(rowss[b], acc.at[sds[b].at[1]], ssems[b], add=True)

        def wait_scatter(b):
            pltpu.make_async_copy(rowss[b], acc.at[sds[b].at[1]], ssems[b]).wait()

        fire_idx(0, 0)
        wait_idx(0, 0)
        fire_gather(0)
        fire_idx(1, 1)

        def outer(t, carry):
            for b in range(3):
                j = 3 * t + b
                nb = (b + 1) % 3
                fb = (b + 2) % 3
                if b < 2:
                    wait_idx(j + 1, nb)
                    fire_gather(nb)
                else:
                    @pl.when(t < nouter - 1)
                    def _():
                        wait_idx(j + 1, nb)
                        fire_gather(nb)
                if b == 0:
                    @pl.when(t > 0)
                    def _():
                        wait_scatter(fb)
                else:
                    wait_scatter(fb)
                if b == 0:
                    fire_idx(j + 2, fb)
                else:
                    @pl.when(t < nouter - 1)
                    def _():
                        fire_idx(j + 2, fb)
                wait_gather(b)
                fire_scatter(b)
            return carry

        lax.fori_loop(0, nouter, outer, 0)
        wait_scatter(2)

        base_x = nouter * 3 * stride + first
        del base_x
        me = s if stride == NS else wid

        @pl.when(me < nxtra)
        def _():
            o = (nouter * 3 * stride + me) * CHUNK
            pltpu.sync_copy(idx_ref.at[:, pl.ds(o, CHUNK)], sd0)
            pltpu.async_copy(table_ref.at[sd0.at[0]], rows0, gsem0).wait()
            pltpu.async_copy(rows0, acc.at[sd0.at[1]], ssem0, add=True)
            pltpu.make_async_copy(rows0, acc.at[sd0.at[1]], ssem0).wait()

    def write_rows(dst_hbm, dst_row0, blks):
        o = 0
        for w in blks:
            pltpu.sync_copy(acc.at[pl.ds(row0 + o, w)],
                            dst_hbm.at[pl.ds(dst_row0 + o, w)])
            o += w

    def write_out(dst_hbm, dst_base):
        @pl.when(s < NS - 1)
        def _():
            write_rows(dst_hbm, dst_base + row0, WBLKS)

        @pl.when(s == NS - 1)
        def _():
            write_rows(dst_hbm, dst_base + row0, OUT_BLKS)

    zero_acc()
    plsc.subcore_barrier()

    @pl.when(c == 0)
    def _():
        run_edges(up, x, s, NS, P1_OUTER, P1_XTRA)

    @pl.when(c == 1)
    def _():
        run_edges(down, x, s, NS, P1_OUTER, P1_XTRA)

    plsc.subcore_barrier()

    @pl.when(c == 0)
    def _():
        write_out(out_up, 0)

    @pl.when(c == 1)
    def _():
        write_out(out_down, 0)

    zero_acc()
    plsc.subcore_barrier()

    run_edges(bnd, battr, wid, NC * NS, P2_OUTER, P2_XTRA)
    plsc.subcore_barrier()
    write_out(pb, c * N)


_sc_call = pl.kernel(
    _sc_body,
    out_type=[
        jax.ShapeDtypeStruct((N, D), jnp.float32),
        jax.ShapeDtypeStruct((N, D), jnp.float32),
        jax.ShapeDtypeStruct((2 * N, D), jnp.float32),
    ],
    mesh=_mesh,
    scratch_types=(
        [pltpu.VMEM((2, CHUNK), jnp.int32)] * 3
        + [pltpu.VMEM((CHUNK, D), jnp.float32)] * 3
        + [pltpu.VMEM_SHARED((NP, D), jnp.float32)]
        + [pltpu.SemaphoreType.DMA] * 9
    ),
)

_BLK = 1000


def _add_body(a_ref, b_ref, o_ref):
    o_ref[...] = a_ref[...] + b_ref[...]


_tc_add = pl.pallas_call(
    _add_body,
    grid=(N // _BLK,),
    in_specs=[
        pl.BlockSpec((_BLK, D), lambda g: (g, 0)),
        pl.BlockSpec((_BLK, D), lambda g: (g + N // _BLK, 0)),
    ],
    out_specs=pl.BlockSpec((_BLK, D), lambda g: (g, 0)),
    out_shape=jax.ShapeDtypeStruct((N, D), jnp.float32),
)


@jax.jit
def kernel(x, up_index, down_index, boundary_index, boundary_attr):
    out_up, out_down, pbp = _sc_call(
        x, boundary_attr, up_index, down_index, boundary_index)
    out_boundary = _tc_add(pbp, pbp)
    return (out_up, out_down, out_boundary)


# speedup vs baseline: 14.0296x; 1.0213x over previous
import functools

import jax
import jax.numpy as jnp
from jax import lax
from jax.experimental import pallas as pl
from jax.experimental.pallas import tpu as pltpu
from jax.experimental.pallas import tpu_sc as plsc

N = 10000
E = 320000
D = 128
NC = 2
NS = 16
CHUNK = 128
NCHUNKS = E // CHUNK
ROWS_PER_TILE = 632
LAST_ROWS = N - 15 * ROWS_PER_TILE
WBLKS = (128, 128, 128, 128, 120)
LBLKS = (128, 128, 128, 128, 8)

NRB = 3
NIB = 6
P1_OUTER = NCHUNKS // NS // NIB
P1_XTRA = NCHUNKS - P1_OUTER * NIB * NS
P2_OUTER = NCHUNKS // (NC * NS) // NIB
P2_XTRA = NCHUNKS - P2_OUTER * NIB * NC * NS

_mesh = plsc.VectorSubcoreMesh(
    core_axis_name="c", subcore_axis_name="s", num_cores=NC, num_subcores=NS)


def _sc_body(x, battr, up, down, bnd,
             out_up, out_down, pb,
             sd0, sd1, sd2, sd3, sd4, sd5, rows0, rows1, rows2, acc,
             isem0, isem1, isem2, isem3, isem4, isem5,
             gsem0, gsem1, gsem2, ssem0, ssem1, ssem2):
    c = lax.axis_index("c")
    s = lax.axis_index("s")
    wid = c * NS + s
    row0 = s * ROWS_PER_TILE
    sds = (sd0, sd1, sd2, sd3, sd4, sd5)
    rowss = (rows0, rows1, rows2)
    isems = (isem0, isem1, isem2, isem3, isem4, isem5)
    gsems = (gsem0, gsem1, gsem2)
    ssems = (ssem0, ssem1, ssem2)

    def tile_blocks(fn):
        @pl.when(s < NS - 1)
        def _():
            o = 0
            for w in WBLKS:
                fn(o, w)
                o += w

        @pl.when(s == NS - 1)
        def _():
            o = 0
            for w in LBLKS:
                fn(o, w)
                o += w

    def zero_acc():
        def zrow(i, carry):
            for k in range(D // 16):
                rows0[i, pl.ds(k * 16, 16)] = jnp.zeros((16,), jnp.float32)
            return carry
        lax.fori_loop(0, CHUNK, zrow, 0)
        tile_blocks(lambda o, w: pltpu.sync_copy(
            rows0.at[pl.ds(0, w)], acc.at[pl.ds(row0 + o, w)]))

    def run_edges(idx_ref, table_ref, first, stride, nouter, nxtra):
        def off(j):
            return (first + j * stride) * CHUNK

        def fire_idx(j, ib):
            pltpu.async_copy(idx_ref.at[:, pl.ds(off(j), CHUNK)], sds[ib], isems[ib])

        def wait_idx(j, ib):
            pltpu.make_async_copy(
                idx_ref.at[:, pl.ds(off(j), CHUNK)], sds[ib], isems[ib]).wait()

        def fire_gather(ib, rb):
            pltpu.async_copy(table_ref.at[sds[ib].at[0]], rowss[rb], gsems[rb])

        def wait_gather(ib, rb):
            pltpu.make_async_copy(table_ref.at[sds[ib].at[0]], rowss[rb], gsems[rb]).wait()

        def fire_scatter(ib, rb):
            pltpu.async_copy(rowss[rb], acc.at[sds[ib].at[1]], ssems[rb], add=True)

        def wait_scatter(ib, rb):
            pltpu.make_async_copy(rowss[rb], acc.at[sds[ib].at[1]], ssems[rb]).wait()

        for j0 in range(4):
            fire_idx(j0, j0)
        wait_idx(0, 0)
        fire_gather(0, 0)

        def outer(t, carry):
            for b in range(NIB):
                j = NIB * t + b
                rb = b % NRB
                if b >= 2:
                    wait_scatter((b - 2) % NIB, (b + 1) % NRB)
                else:
                    @pl.when(t > 0)
                    def _():
                        wait_scatter((b - 2) % NIB, (b + 1) % NRB)
                if b < NIB - 1:
                    wait_idx(j + 1, (b + 1) % NIB)
                    fire_gather((b + 1) % NIB, (b + 1) % NRB)
                else:
                    @pl.when(t < nouter - 1)
                    def _():
                        wait_idx(j + 1, 0)
                        fire_gather(0, 0)
                if b < 2:
                    fire_idx(j + 4, (b + 4) % NIB)
                else:
                    @pl.when(t < nouter - 1)
                    def _():
                        fire_idx(j + 4, (b + 4) % NIB)
                wait_gather(b, rb)
                fire_scatter(b, rb)
            return carry

        lax.fori_loop(0, nouter, outer, 0)
        wait_scatter(NIB - 2, (NIB - 2) % NRB)
        wait_scatter(NIB - 1, (NIB - 1) % NRB)

        me = s if stride == NS else wid

        @pl.when(me < nxtra)
        def _():
            o = (nouter * NIB * stride + me) * CHUNK
            pltpu.sync_copy(idx_ref.at[:, pl.ds(o, CHUNK)], sd0)
            pltpu.async_copy(table_ref.at[sd0.at[0]], rows0, gsem0).wait()
            pltpu.async_copy(rows0, acc.at[sd0.at[1]], ssem0, add=True)
            pltpu.make_async_copy(rows0, acc.at[sd0.at[1]], ssem0).wait()

    def write_out(dst_hbm, dst_base):
        tile_blocks(lambda o, w: pltpu.sync_copy(
            acc.at[pl.ds(row0 + o, w)],
            dst_hbm.at[pl.ds(dst_base + row0 + o, w)]))

    zero_acc()
    plsc.subcore_barrier()

    @pl.when(c == 0)
    def _():
        run_edges(up, x, s, NS, P1_OUTER, P1_XTRA)

    @pl.when(c == 1)
    def _():
        run_edges(down, x, s, NS, P1_OUTER, P1_XTRA)

    plsc.subcore_barrier()

    @pl.when(c == 0)
    def _():
        write_out(out_up, 0)

    @pl.when(c == 1)
    def _():
        write_out(out_down, 0)

    zero_acc()
    plsc.subcore_barrier()

    run_edges(bnd, battr, wid, NC * NS, P2_OUTER, P2_XTRA)
    plsc.subcore_barrier()
    write_out(pb, c * N)


_sc_call = pl.kernel(
    _sc_body,
    out_type=[
        jax.ShapeDtypeStruct((N, D), jnp.float32),
        jax.ShapeDtypeStruct((N, D), jnp.float32),
        jax.ShapeDtypeStruct((2 * N, D), jnp.float32),
    ],
    mesh=_mesh,
    scratch_types=(
        [pltpu.VMEM((2, CHUNK), jnp.int32)] * 6
        + [pltpu.VMEM((CHUNK, D), jnp.float32)] * 3
        + [pltpu.VMEM_SHARED((N, D), jnp.float32)]
        + [pltpu.SemaphoreType.DMA] * 12
    ),
)

_BLK = 1000


def _add_body(a_ref, b_ref, o_ref):
    o_ref[...] = a_ref[...] + b_ref[...]


_tc_add = pl.pallas_call(
    _add_body,
    grid=(N // _BLK,),
    in_specs=[
        pl.BlockSpec((_BLK, D), lambda g: (g, 0)),
        pl.BlockSpec((_BLK, D), lambda g: (g + N // _BLK, 0)),
    ],
    out_specs=pl.BlockSpec((_BLK, D), lambda g: (g, 0)),
    out_shape=jax.ShapeDtypeStruct((N, D), jnp.float32),
)


@jax.jit
def kernel(x, up_index, down_index, boundary_index, boundary_attr):
    out_up, out_down, pbp = _sc_call(
        x, boundary_attr, up_index, down_index, boundary_index)
    out_boundary = _tc_add(pbp, pbp)
    return (out_up, out_down, out_boundary)


# speedup vs baseline: 14.2276x; 1.0141x over previous
import functools

import jax
import jax.numpy as jnp
from jax import lax
from jax.experimental import pallas as pl
from jax.experimental.pallas import tpu as pltpu
from jax.experimental.pallas import tpu_sc as plsc

N = 10000
E = 320000
D = 128
NC = 2
NS = 16
CHUNK = 128
NCHUNKS = E // CHUNK
ROWS_PER_TILE = 632
WBLKS = (128, 128, 128, 128, 120)
LBLKS = (128, 128, 128, 128, 8)

NRB = 3
NIB = 6
P1_OUTER = NCHUNKS // NS // NIB
P1_XTRA = NCHUNKS - P1_OUTER * NIB * NS
P2_OUTER = NCHUNKS // (NC * NS) // NIB
P2_XTRA = NCHUNKS - P2_OUTER * NIB * NC * NS

_mesh = plsc.VectorSubcoreMesh(
    core_axis_name="c", subcore_axis_name="s", num_cores=NC, num_subcores=NS)

_SCRATCH = (
    [pltpu.VMEM((2, CHUNK), jnp.int32)] * 6
    + [pltpu.VMEM((CHUNK, D), jnp.float32)] * 3
    + [pltpu.VMEM_SHARED((N, D), jnp.float32)]
    + [pltpu.SemaphoreType.DMA] * 12
)


def _make_helpers(scratch):
    (sd0, sd1, sd2, sd3, sd4, sd5, rows0, rows1, rows2, acc,
     isem0, isem1, isem2, isem3, isem4, isem5,
     gsem0, gsem1, gsem2, ssem0, ssem1, ssem2) = scratch
    c = lax.axis_index("c")
    s = lax.axis_index("s")
    wid = c * NS + s
    row0 = s * ROWS_PER_TILE
    sds = (sd0, sd1, sd2, sd3, sd4, sd5)
    rowss = (rows0, rows1, rows2)
    isems = (isem0, isem1, isem2, isem3, isem4, isem5)
    gsems = (gsem0, gsem1, gsem2)
    ssems = (ssem0, ssem1, ssem2)

    def tile_blocks(fn):
        @pl.when(s < NS - 1)
        def _():
            o = 0
            for w in WBLKS:
                fn(o, w)
                o += w

        @pl.when(s == NS - 1)
        def _():
            o = 0
            for w in LBLKS:
                fn(o, w)
                o += w

    def zero_acc():
        def zrow(i, carry):
            for k in range(D // 16):
                rows0[i, pl.ds(k * 16, 16)] = jnp.zeros((16,), jnp.float32)
            return carry
        lax.fori_loop(0, CHUNK, zrow, 0)
        tile_blocks(lambda o, w: pltpu.sync_copy(
            rows0.at[pl.ds(0, w)], acc.at[pl.ds(row0 + o, w)]))

    def run_edges(idx_ref, table_ref, first, stride, nouter, nxtra):
        def off(j):
            return (first + j * stride) * CHUNK

        def fire_idx(j, ib):
            pltpu.async_copy(idx_ref.at[:, pl.ds(off(j), CHUNK)], sds[ib], isems[ib])

        def wait_idx(j, ib):
            pltpu.make_async_copy(
                idx_ref.at[:, pl.ds(off(j), CHUNK)], sds[ib], isems[ib]).wait()

        def fire_gather(ib, rb):
            pltpu.async_copy(table_ref.at[sds[ib].at[0]], rowss[rb], gsems[rb])

        def wait_gather(ib, rb):
            pltpu.make_async_copy(table_ref.at[sds[ib].at[0]], rowss[rb], gsems[rb]).wait()

        def fire_scatter(ib, rb):
            pltpu.async_copy(rowss[rb], acc.at[sds[ib].at[1]], ssems[rb], add=True)

        def wait_scatter(ib, rb):
            pltpu.make_async_copy(rowss[rb], acc.at[sds[ib].at[1]], ssems[rb]).wait()

        for j0 in range(4):
            fire_idx(j0, j0)
        wait_idx(0, 0)
        fire_gather(0, 0)

        def outer(t, carry):
            for b in range(NIB):
                j = NIB * t + b
                rb = b % NRB
                if b >= 2:
                    wait_scatter((b - 2) % NIB, (b + 1) % NRB)
                else:
                    @pl.when(t > 0)
                    def _():
                        wait_scatter((b - 2) % NIB, (b + 1) % NRB)
                if b < NIB - 1:
                    wait_idx(j + 1, (b + 1) % NIB)
                    fire_gather((b + 1) % NIB, (b + 1) % NRB)
                else:
                    @pl.when(t < nouter - 1)
                    def _():
                        wait_idx(j + 1, 0)
                        fire_gather(0, 0)
                if b < 2:
                    fire_idx(j + 4, (b + 4) % NIB)
                else:
                    @pl.when(t < nouter - 1)
                    def _():
                        fire_idx(j + 4, (b + 4) % NIB)
                wait_gather(b, rb)
                fire_scatter(b, rb)
            return carry

        lax.fori_loop(0, nouter, outer, 0)
        wait_scatter(NIB - 2, (NIB - 2) % NRB)
        wait_scatter(NIB - 1, (NIB - 1) % NRB)

        me = s if stride == NS else wid

        @pl.when(me < nxtra)
        def _():
            o = (nouter * NIB * stride + me) * CHUNK
            pltpu.sync_copy(idx_ref.at[:, pl.ds(o, CHUNK)], sd0)
            pltpu.async_copy(table_ref.at[sd0.at[0]], rows0, gsem0).wait()
            pltpu.async_copy(rows0, acc.at[sd0.at[1]], ssem0, add=True)
            pltpu.make_async_copy(rows0, acc.at[sd0.at[1]], ssem0).wait()

    def write_out(dst_hbm, dst_base):
        tile_blocks(lambda o, w: pltpu.sync_copy(
            acc.at[pl.ds(row0 + o, w)],
            dst_hbm.at[pl.ds(dst_base + row0 + o, w)]))

    return c, s, wid, zero_acc, run_edges, write_out


def _ud_body(x, up, down, out_up, out_down, *scratch):
    c, s, wid, zero_acc, run_edges, write_out = _make_helpers(scratch)
    zero_acc()
    plsc.subcore_barrier()

    @pl.when(c == 0)
    def _():
        run_edges(up, x, s, NS, P1_OUTER, P1_XTRA)

    @pl.when(c == 1)
    def _():
        run_edges(down, x, s, NS, P1_OUTER, P1_XTRA)

    plsc.subcore_barrier()

    @pl.when(c == 0)
    def _():
        write_out(out_up, 0)

    @pl.when(c == 1)
    def _():
        write_out(out_down, 0)


def _b_body(battr, bnd, pb, *scratch):
    c, s, wid, zero_acc, run_edges, write_out = _make_helpers(scratch)
    zero_acc()
    plsc.subcore_barrier()
    run_edges(bnd, battr, wid, NC * NS, P2_OUTER, P2_XTRA)
    plsc.subcore_barrier()
    write_out(pb, c * N)


_ud_call = pl.kernel(
    _ud_body,
    out_type=[
        jax.ShapeDtypeStruct((N, D), jnp.float32),
        jax.ShapeDtypeStruct((N, D), jnp.float32),
    ],
    mesh=_mesh,
    scratch_types=list(_SCRATCH),
)

_b_call = pl.kernel(
    _b_body,
    out_type=[jax.ShapeDtypeStruct((2 * N, D), jnp.float32)],
    mesh=_mesh,
    scratch_types=list(_SCRATCH),
)

_BLK = 1000


def _add_body(a_ref, b_ref, o_ref):
    o_ref[...] = a_ref[...] + b_ref[...]


_tc_add = pl.pallas_call(
    _add_body,
    grid=(N // _BLK,),
    in_specs=[
        pl.BlockSpec((_BLK, D), lambda g: (g, 0)),
        pl.BlockSpec((_BLK, D), lambda g: (g + N // _BLK, 0)),
    ],
    out_specs=pl.BlockSpec((_BLK, D), lambda g: (g, 0)),
    out_shape=jax.ShapeDtypeStruct((N, D), jnp.float32),
)


@jax.jit
def kernel(x, up_index, down_index, boundary_index, boundary_attr):
    (pbp,) = _b_call(boundary_attr, boundary_index)
    out_boundary = _tc_add(pbp, pbp)
    out_up, out_down = _ud_call(x, up_index, down_index)
    return (out_up, out_down, out_boundary)
